# baseline (speedup 1.0000x reference)
import functools

import jax
import jax.numpy as jnp
from jax import lax
from jax.experimental import pallas as pl
from jax.experimental.pallas import tpu as pltpu
from jax.experimental.pallas import tpu_sc as plsc

N = 100000
IN_C = 64
HID = 512
OUT_C = 13
GRID = 0.1

BLK = 1024
NB = 98
NPAD = NB * BLK
NTILE = 16
PT = NPAD // NTILE
CH = 128
NCHUNK = PT // CH
CBW = 16
NCB_PER_CORE = (HID // CBW) // 2
ZR = 256


def _stats1_body(feat_ref, w1t_ref, acc_ref):
    i = pl.program_id(0)
    x = jnp.dot(feat_ref[...], w1t_ref[...], preferred_element_type=jnp.float32)
    s = jnp.sum(x, axis=0, keepdims=True)
    sq = jnp.sum(x * x, axis=0, keepdims=True)

    @pl.when(i == 0)
    def _():
        acc_ref[...] = jnp.zeros_like(acc_ref)

    acc_ref[0:1, :] += s
    acc_ref[1:2, :] += sq


def _fwd1_body(feat_ref, c0_ref, c1_ref, c2_ref, w1t_ref, a1_ref, b1_ref,
               y_ref, idx_ref, vm_ref):
    i = pl.program_id(0)
    x = jnp.dot(feat_ref[...], w1t_ref[...], preferred_element_type=jnp.float32)
    y = jnp.maximum(x * a1_ref[...] + b1_ref[...], 0.0)
    rows = i * BLK + lax.broadcasted_iota(jnp.int32, (BLK, 1), 0)
    y_ref[...] = jnp.where(rows < N, y, 0.0)

    lanes = i * BLK + lax.broadcasted_iota(jnp.int32, (1, 1, BLK), 2)
    v0 = jnp.floor(c0_ref[...] / GRID).astype(jnp.int32)
    v1 = jnp.floor(c1_ref[...] / GRID).astype(jnp.int32)
    v2 = jnp.floor(c2_ref[...] / GRID).astype(jnp.int32)
    key = v0 * 10000 + v1 * 100 + v2
    idx = jnp.clip(jnp.mod(key, N), 0, N - 1)
    valid = lanes < N
    idx_ref[...] = jnp.where(valid, idx, 0)
    vm_ref[...] = jnp.where(valid, 1.0, 0.0)


def _fwd2_body(ps_ref, cnt_ref, w2t_ref, h_ref, acc_ref):
    i = pl.program_id(0)
    cnt = jnp.maximum(cnt_ref[...], 1.0)
    p = ps_ref[...] / cnt
    h = jnp.dot(p, w2t_ref[...], preferred_element_type=jnp.float32)
    h_ref[...] = h
    s = jnp.sum(h, axis=0, keepdims=True)
    sq = jnp.sum(h * h, axis=0, keepdims=True)

    @pl.when(i == 0)
    def _():
        acc_ref[...] = jnp.zeros_like(acc_ref)

    acc_ref[0:1, :] += s
    acc_ref[1:2, :] += sq


def _out_body(h_ref, a2_ref, b2_ref, wot_ref, bout_ref, o_ref):
    y2 = jnp.maximum(h_ref[...] * a2_ref[...] + b2_ref[...], 0.0)
    o_ref[...] = (jnp.dot(y2, wot_ref[...], preferred_element_type=jnp.float32)
                  + bout_ref[...])


def _full(shape):
    return pl.BlockSpec(shape, lambda i: (0,) * len(shape))


def _stats1(feat_pad, w1t):
    return pl.pallas_call(
        _stats1_body,
        grid=(NB,),
        in_specs=[pl.BlockSpec((BLK, IN_C), lambda i: (i, 0)), _full((IN_C, HID))],
        out_specs=_full((8, HID)),
        out_shape=jax.ShapeDtypeStruct((8, HID), jnp.float32),
    )(feat_pad, w1t)


def _fwd1(feat_pad, c0, c1, c2, w1t, a1, b1):
    return pl.pallas_call(
        _fwd1_body,
        grid=(NB,),
        in_specs=[
            pl.BlockSpec((BLK, IN_C), lambda i: (i, 0)),
            pl.BlockSpec((1, 1, BLK), lambda i: (i, 0, 0)),
            pl.BlockSpec((1, 1, BLK), lambda i: (i, 0, 0)),
            pl.BlockSpec((1, 1, BLK), lambda i: (i, 0, 0)),
            _full((IN_C, HID)),
            _full((1, HID)),
            _full((1, HID)),
        ],
        out_specs=[
            pl.BlockSpec((BLK, HID), lambda i: (i, 0)),
            pl.BlockSpec((1, 1, BLK), lambda i: (i, 0, 0)),
            pl.BlockSpec((1, 1, BLK), lambda i: (i, 0, 0)),
        ],
        out_shape=[
            jax.ShapeDtypeStruct((NPAD, HID), jnp.float32),
            jax.ShapeDtypeStruct((NB, 1, BLK), jnp.int32),
            jax.ShapeDtypeStruct((NB, 1, BLK), jnp.float32),
        ],
    )(feat_pad, c0, c1, c2, w1t, a1, b1)


def _fwd2(pooled, counts2d, w2t):
    return pl.pallas_call(
        _fwd2_body,
        grid=(NB,),
        in_specs=[
            pl.BlockSpec((BLK, HID), lambda i: (i, 0)),
            pl.BlockSpec((BLK, 1), lambda i: (i, 0)),
            _full((HID, HID)),
        ],
        out_specs=[
            pl.BlockSpec((BLK, HID), lambda i: (i, 0)),
            _full((8, HID)),
        ],
        out_shape=[
            jax.ShapeDtypeStruct((NPAD, HID), jnp.float32),
            jax.ShapeDtypeStruct((8, HID), jnp.float32),
        ],
    )(pooled, counts2d, w2t)


def _out(h, a2, b2, wot, bout2):
    return pl.pallas_call(
        _out_body,
        grid=(NB,),
        in_specs=[
            pl.BlockSpec((BLK, HID), lambda i: (i, 0)),
            _full((1, HID)),
            _full((1, HID)),
            _full((HID, OUT_C)),
            _full((1, OUT_C)),
        ],
        out_specs=pl.BlockSpec((BLK, OUT_C), lambda i: (i, 0)),
        out_shape=jax.ShapeDtypeStruct((NPAD, OUT_C), jnp.float32),
    )(h, a2, b2, wot, bout2)


def _sc_body(y_hbm, idx_hbm, vm_hbm, z_hbm, z1_hbm,
             pooled_hbm, counts_hbm,
             acc, cacc, ibuf, ybuf, zbuf, vbuf):
    c = lax.axis_index("c")
    s = lax.axis_index("s")
    base = s * PT

    pltpu.sync_copy(idx_hbm.at[s], ibuf)
    pltpu.sync_copy(z_hbm, zbuf)

    @pl.when(c == 0)
    def _():
        pltpu.sync_copy(vm_hbm.at[s], vbuf)

    def _zero_acc():
        for z in range(PT // ZR):
            pltpu.sync_copy(zbuf, acc.at[pl.ds(base + z * ZR, ZR), :])
        pltpu.sync_copy(zbuf.at[pl.ds(0, CH), :],
                        acc.at[pl.ds(base + (PT // ZR) * ZR, CH), :])

    for j in range(NCB_PER_CORE):
        col = (c * NCB_PER_CORE + j) * CBW
        _zero_acc()
        plsc.subcore_barrier()

        def _chunk(k, carry):
            r0 = base + k * CH
            pltpu.sync_copy(y_hbm.at[pl.ds(r0, CH), pl.ds(col, CBW)], ybuf)
            pltpu.sync_copy(ybuf, acc.at[ibuf.at[k]], add=True)
            return carry

        lax.fori_loop(0, NCHUNK, _chunk, 0)
        plsc.subcore_barrier()

        for z in range(6):
            pltpu.sync_copy(acc.at[pl.ds(base + z * BLK, BLK), :],
                            pooled_hbm.at[pl.ds(base + z * BLK, BLK),
                                          pl.ds(col, CBW)])
        pltpu.sync_copy(acc.at[pl.ds(base + 6 * BLK, CH), :],
                        pooled_hbm.at[pl.ds(base + 6 * BLK, CH),
                                      pl.ds(col, CBW)])

    @pl.when(c == 0)
    def _():
        for z in range(6):
            pltpu.sync_copy(z1_hbm, cacc.at[pl.ds(base + z * BLK, BLK)])
        pltpu.sync_copy(z1_hbm.at[pl.ds(0, CH)],
                        cacc.at[pl.ds(base + 6 * BLK, CH)])
        plsc.subcore_barrier()

        def _cchunk(k, carry):
            pltpu.sync_copy(vbuf.at[k], cacc.at[ibuf.at[k]], add=True)
            return carry

        lax.fori_loop(0, NCHUNK, _cchunk, 0)
        plsc.subcore_barrier()
        pltpu.sync_copy(cacc.at[pl.ds(base, PT)], counts_hbm.at[pl.ds(base, PT)])


def _sc_scatter(y, idx3d, vm3d, zeros2d, zeros1d):
    kern = functools.partial(
        pl.kernel,
        out_type=[
            jax.ShapeDtypeStruct((NPAD, HID), jnp.float32),
            jax.ShapeDtypeStruct((NPAD,), jnp.float32),
        ],
        mesh=plsc.VectorSubcoreMesh(core_axis_name="c", subcore_axis_name="s"),
        compiler_params=pltpu.CompilerParams(use_tc_tiling_on_sc=False),
        scratch_types=[
            pltpu.VMEM_SHARED((NPAD, CBW), jnp.float32),
            pltpu.VMEM_SHARED((NPAD,), jnp.float32),
            pltpu.VMEM((NCHUNK, CH), jnp.int32),
            pltpu.VMEM((CH, CBW), jnp.float32),
            pltpu.VMEM((ZR, CBW), jnp.float32),
            pltpu.VMEM((NCHUNK, CH), jnp.float32),
        ],
    )(_sc_body)
    return kern(y, idx3d, vm3d, zeros2d, zeros1d)


def kernel(feat, coord, offset, W1, gamma1, beta1, W2, gamma2, beta2, Wout, bout):
    del offset
    f32 = jnp.float32
    pad = NPAD - N
    feat_pad = jnp.pad(feat, ((0, pad), (0, 0)))
    c0 = jnp.pad(coord[:, 0], (0, pad)).reshape(NB, 1, BLK)
    c1 = jnp.pad(coord[:, 1], (0, pad)).reshape(NB, 1, BLK)
    c2 = jnp.pad(coord[:, 2], (0, pad)).reshape(NB, 1, BLK)
    w1t = W1.T
    w2t = W2.T
    wot = Wout.T
    eps = 1e-5

    st1 = _stats1(feat_pad, w1t)
    mean1 = st1[0] / N
    var1 = st1[1] / N - mean1 * mean1
    a1 = gamma1 / jnp.sqrt(var1 + eps)
    b1 = beta1 - mean1 * a1

    y, idx3, vm3 = _fwd1(feat_pad, c0, c1, c2, w1t,
                         a1.reshape(1, HID).astype(f32),
                         b1.reshape(1, HID).astype(f32))

    idx3d = idx3.reshape(NTILE, NCHUNK, CH)
    vm3d = vm3.reshape(NTILE, NCHUNK, CH)
    zeros2d = jnp.zeros((ZR, CBW), f32)
    zeros1d = jnp.zeros((BLK,), f32)

    pooled, counts = _sc_scatter(y, idx3d, vm3d, zeros2d, zeros1d)

    h, st2 = _fwd2(pooled, counts.reshape(NPAD, 1), w2t)
    mean2 = st2[0] / N
    var2 = st2[1] / N - mean2 * mean2
    a2 = gamma2 / jnp.sqrt(var2 + eps)
    b2 = beta2 - mean2 * a2

    logits = _out(h, a2.reshape(1, HID).astype(f32),
                  b2.reshape(1, HID).astype(f32),
                  wot, bout.reshape(1, OUT_C))
    return logits[:N]


# speedup vs baseline: 1.1764x; 1.1764x over previous
import functools

import jax
import jax.numpy as jnp
from jax import lax
from jax.experimental import pallas as pl
from jax.experimental.pallas import tpu as pltpu
from jax.experimental.pallas import tpu_sc as plsc

N = 100000
IN_C = 64
HID = 512
OUT_C = 13
GRID = 0.1

BLK = 1024
NB = 98
NPAD = NB * BLK
NTILE = 16
PT = NPAD // NTILE
CH = 128
NCHUNK = PT // CH
CBW = 16
NCB_PER_CORE = (HID // CBW) // 2
ZR = 256


def _stats1_body(feat_ref, w1t_ref, acc_ref):
    i = pl.program_id(0)
    x = jnp.dot(feat_ref[...], w1t_ref[...], preferred_element_type=jnp.float32)
    s = jnp.sum(x, axis=0, keepdims=True)
    sq = jnp.sum(x * x, axis=0, keepdims=True)

    @pl.when(i == 0)
    def _():
        acc_ref[...] = jnp.zeros_like(acc_ref)

    acc_ref[0:1, :] += s
    acc_ref[1:2, :] += sq


def _fwd1_body(feat_ref, c0_ref, c1_ref, c2_ref, w1t_ref, a1_ref, b1_ref,
               y_ref, idx_ref, vm_ref):
    i = pl.program_id(0)
    x = jnp.dot(feat_ref[...], w1t_ref[...], preferred_element_type=jnp.float32)
    y = jnp.maximum(x * a1_ref[...] + b1_ref[...], 0.0)
    rows = i * BLK + lax.broadcasted_iota(jnp.int32, (BLK, 1), 0)
    y_ref[...] = jnp.where(rows < N, y, 0.0)

    lanes = i * BLK + lax.broadcasted_iota(jnp.int32, (1, 1, BLK), 2)
    v0 = jnp.floor(c0_ref[...] / GRID).astype(jnp.int32)
    v1 = jnp.floor(c1_ref[...] / GRID).astype(jnp.int32)
    v2 = jnp.floor(c2_ref[...] / GRID).astype(jnp.int32)
    key = v0 * 10000 + v1 * 100 + v2
    idx = jnp.clip(jnp.mod(key, N), 0, N - 1)
    valid = lanes < N
    idx_ref[...] = jnp.where(valid, idx, 0)
    vm_ref[...] = jnp.where(valid, 1.0, 0.0)


def _fwd2_body(ps_ref, cnt_ref, w2t_ref, h_ref, acc_ref):
    i = pl.program_id(0)
    cnt = jnp.maximum(cnt_ref[...], 1.0)
    p = ps_ref[...] / cnt
    h = jnp.dot(p, w2t_ref[...], preferred_element_type=jnp.float32)
    h_ref[...] = h
    s = jnp.sum(h, axis=0, keepdims=True)
    sq = jnp.sum(h * h, axis=0, keepdims=True)

    @pl.when(i == 0)
    def _():
        acc_ref[...] = jnp.zeros_like(acc_ref)

    acc_ref[0:1, :] += s
    acc_ref[1:2, :] += sq


def _out_body(h_ref, a2_ref, b2_ref, wot_ref, bout_ref, o_ref):
    y2 = jnp.maximum(h_ref[...] * a2_ref[...] + b2_ref[...], 0.0)
    o_ref[...] = (jnp.dot(y2, wot_ref[...], preferred_element_type=jnp.float32)
                  + bout_ref[...])


def _full(shape):
    return pl.BlockSpec(shape, lambda i: (0,) * len(shape))


def _stats1(feat_pad, w1t):
    return pl.pallas_call(
        _stats1_body,
        grid=(NB,),
        in_specs=[pl.BlockSpec((BLK, IN_C), lambda i: (i, 0)), _full((IN_C, HID))],
        out_specs=_full((8, HID)),
        out_shape=jax.ShapeDtypeStruct((8, HID), jnp.float32),
    )(feat_pad, w1t)


def _fwd1(feat_pad, c0, c1, c2, w1t, a1, b1):
    return pl.pallas_call(
        _fwd1_body,
        grid=(NB,),
        in_specs=[
            pl.BlockSpec((BLK, IN_C), lambda i: (i, 0)),
            pl.BlockSpec((1, 1, BLK), lambda i: (i, 0, 0)),
            pl.BlockSpec((1, 1, BLK), lambda i: (i, 0, 0)),
            pl.BlockSpec((1, 1, BLK), lambda i: (i, 0, 0)),
            _full((IN_C, HID)),
            _full((1, HID)),
            _full((1, HID)),
        ],
        out_specs=[
            pl.BlockSpec((BLK, HID), lambda i: (i, 0)),
            pl.BlockSpec((1, 1, BLK), lambda i: (i, 0, 0)),
            pl.BlockSpec((1, 1, BLK), lambda i: (i, 0, 0)),
        ],
        out_shape=[
            jax.ShapeDtypeStruct((NPAD, HID), jnp.float32),
            jax.ShapeDtypeStruct((NB, 1, BLK), jnp.int32),
            jax.ShapeDtypeStruct((NB, 1, BLK), jnp.float32),
        ],
    )(feat_pad, c0, c1, c2, w1t, a1, b1)


def _fwd2(pooled, counts2d, w2t):
    return pl.pallas_call(
        _fwd2_body,
        grid=(NB,),
        in_specs=[
            pl.BlockSpec((BLK, HID), lambda i: (i, 0)),
            pl.BlockSpec((BLK, 1), lambda i: (i, 0)),
            _full((HID, HID)),
        ],
        out_specs=[
            pl.BlockSpec((BLK, HID), lambda i: (i, 0)),
            _full((8, HID)),
        ],
        out_shape=[
            jax.ShapeDtypeStruct((NPAD, HID), jnp.float32),
            jax.ShapeDtypeStruct((8, HID), jnp.float32),
        ],
    )(pooled, counts2d, w2t)


def _out(h, a2, b2, wot, bout2):
    return pl.pallas_call(
        _out_body,
        grid=(NB,),
        in_specs=[
            pl.BlockSpec((BLK, HID), lambda i: (i, 0)),
            _full((1, HID)),
            _full((1, HID)),
            _full((HID, OUT_C)),
            _full((1, OUT_C)),
        ],
        out_specs=pl.BlockSpec((BLK, OUT_C), lambda i: (i, 0)),
        out_shape=jax.ShapeDtypeStruct((NPAD, OUT_C), jnp.float32),
    )(h, a2, b2, wot, bout2)


def _sc_body(y_hbm, idx_hbm, vm_hbm, z_hbm, z1_hbm,
             pooled_hbm, counts_hbm,
             acc, cacc, ibuf, ybuf0, ybuf1, zbuf, vbuf, rs0, rs1, ws):
    c = lax.axis_index("c")
    s = lax.axis_index("s")
    base = s * PT

    pltpu.sync_copy(idx_hbm.at[s], ibuf)
    pltpu.sync_copy(z_hbm, zbuf)

    @pl.when(c == 0)
    def _():
        pltpu.sync_copy(vm_hbm.at[s], vbuf)

    for j in range(NCB_PER_CORE):
        col = (c * NCB_PER_CORE + j) * CBW

        def _rd(k, buf, sem, col=col):
            return pltpu.async_copy(
                y_hbm.at[pl.ds(base + k * CH, CH), pl.ds(col, CBW)],
                buf, sem)

        _rd(0, ybuf0, rs0)
        _rd(1, ybuf1, rs1)
        for z in range(PT // ZR):
            pltpu.sync_copy(zbuf, acc.at[pl.ds(base + z * ZR, ZR), :])
        pltpu.sync_copy(zbuf.at[pl.ds(0, CH), :],
                        acc.at[pl.ds(base + (PT // ZR) * ZR, CH), :])
        plsc.subcore_barrier()

        def _pair(u, carry):
            k0 = 2 * u
            pltpu.make_async_copy(
                y_hbm.at[pl.ds(base + k0 * CH, CH), pl.ds(col, CBW)],
                ybuf0, rs0).wait()
            pltpu.sync_copy(ybuf0, acc.at[ibuf.at[k0]], add=True)
            _rd(k0 + 2, ybuf0, rs0)
            pltpu.make_async_copy(
                y_hbm.at[pl.ds(base + (k0 + 1) * CH, CH), pl.ds(col, CBW)],
                ybuf1, rs1).wait()
            pltpu.sync_copy(ybuf1, acc.at[ibuf.at[k0 + 1]], add=True)

            @pl.when(k0 + 3 < NCHUNK)
            def _():
                _rd(k0 + 3, ybuf1, rs1)

            return carry

        lax.fori_loop(0, (NCHUNK - 1) // 2, _pair, 0)
        pltpu.make_async_copy(
            y_hbm.at[pl.ds(base + (NCHUNK - 1) * CH, CH), pl.ds(col, CBW)],
            ybuf0, rs0).wait()
        pltpu.sync_copy(ybuf0, acc.at[ibuf.at[NCHUNK - 1]], add=True)
        plsc.subcore_barrier()

        for z in range(6):
            pltpu.async_copy(acc.at[pl.ds(base + z * BLK, BLK), :],
                             pooled_hbm.at[pl.ds(base + z * BLK, BLK),
                                           pl.ds(col, CBW)], ws)
        pltpu.async_copy(acc.at[pl.ds(base + 6 * BLK, CH), :],
                         pooled_hbm.at[pl.ds(base + 6 * BLK, CH),
                                       pl.ds(col, CBW)], ws)
        for z in range(6):
            pltpu.make_async_copy(acc.at[pl.ds(base + z * BLK, BLK), :],
                                  pooled_hbm.at[pl.ds(base + z * BLK, BLK),
                                                pl.ds(col, CBW)], ws).wait()
        pltpu.make_async_copy(acc.at[pl.ds(base + 6 * BLK, CH), :],
                              pooled_hbm.at[pl.ds(base + 6 * BLK, CH),
                                            pl.ds(col, CBW)], ws).wait()

    @pl.when(c == 0)
    def _():
        for z in range(6):
            pltpu.sync_copy(z1_hbm, cacc.at[pl.ds(base + z * BLK, BLK)])
        pltpu.sync_copy(z1_hbm.at[pl.ds(0, CH)],
                        cacc.at[pl.ds(base + 6 * BLK, CH)])
        plsc.subcore_barrier()

        def _cchunk(k, carry):
            pltpu.sync_copy(vbuf.at[k], cacc.at[ibuf.at[k]], add=True)
            return carry

        lax.fori_loop(0, NCHUNK, _cchunk, 0)
        plsc.subcore_barrier()
        pltpu.sync_copy(cacc.at[pl.ds(base, PT)], counts_hbm.at[pl.ds(base, PT)])


def _sc_scatter(y, idx3d, vm3d, zeros2d, zeros1d):
    kern = functools.partial(
        pl.kernel,
        out_type=[
            jax.ShapeDtypeStruct((NPAD, HID), jnp.float32),
            jax.ShapeDtypeStruct((NPAD,), jnp.float32),
        ],
        mesh=plsc.VectorSubcoreMesh(core_axis_name="c", subcore_axis_name="s"),
        compiler_params=pltpu.CompilerParams(use_tc_tiling_on_sc=False),
        scratch_types=[
            pltpu.VMEM_SHARED((NPAD, CBW), jnp.float32),
            pltpu.VMEM_SHARED((NPAD,), jnp.float32),
            pltpu.VMEM((NCHUNK, CH), jnp.int32),
            pltpu.VMEM((CH, CBW), jnp.float32),
            pltpu.VMEM((CH, CBW), jnp.float32),
            pltpu.VMEM((ZR, CBW), jnp.float32),
            pltpu.VMEM((NCHUNK, CH), jnp.float32),
            pltpu.SemaphoreType.DMA,
            pltpu.SemaphoreType.DMA,
            pltpu.SemaphoreType.DMA,
        ],
    )(_sc_body)
    return kern(y, idx3d, vm3d, zeros2d, zeros1d)


def kernel(feat, coord, offset, W1, gamma1, beta1, W2, gamma2, beta2, Wout, bout):
    del offset
    f32 = jnp.float32
    pad = NPAD - N
    feat_pad = jnp.pad(feat, ((0, pad), (0, 0)))
    c0 = jnp.pad(coord[:, 0], (0, pad)).reshape(NB, 1, BLK)
    c1 = jnp.pad(coord[:, 1], (0, pad)).reshape(NB, 1, BLK)
    c2 = jnp.pad(coord[:, 2], (0, pad)).reshape(NB, 1, BLK)
    w1t = W1.T
    w2t = W2.T
    wot = Wout.T
    eps = 1e-5

    st1 = _stats1(feat_pad, w1t)
    mean1 = st1[0] / N
    var1 = st1[1] / N - mean1 * mean1
    a1 = gamma1 / jnp.sqrt(var1 + eps)
    b1 = beta1 - mean1 * a1

    y, idx3, vm3 = _fwd1(feat_pad, c0, c1, c2, w1t,
                         a1.reshape(1, HID).astype(f32),
                         b1.reshape(1, HID).astype(f32))

    idx3d = idx3.reshape(NTILE, NCHUNK, CH)
    vm3d = vm3.reshape(NTILE, NCHUNK, CH)
    zeros2d = jnp.zeros((ZR, CBW), f32)
    zeros1d = jnp.zeros((BLK,), f32)

    pooled, counts = _sc_scatter(y, idx3d, vm3d, zeros2d, zeros1d)

    h, st2 = _fwd2(pooled, counts.reshape(NPAD, 1), w2t)
    mean2 = st2[0] / N
    var2 = st2[1] / N - mean2 * mean2
    a2 = gamma2 / jnp.sqrt(var2 + eps)
    b2 = beta2 - mean2 * a2

    logits = _out(h, a2.reshape(1, HID).astype(f32),
                  b2.reshape(1, HID).astype(f32),
                  wot, bout.reshape(1, OUT_C))
    return logits[:N]


# speedup vs baseline: 1.2020x; 1.0218x over previous
import functools

import jax
import jax.numpy as jnp
from jax import lax
from jax.experimental import pallas as pl
from jax.experimental.pallas import tpu as pltpu
from jax.experimental.pallas import tpu_sc as plsc

N = 100000
IN_C = 64
HID = 512
OUT_C = 13
GRID = 0.1

BLK = 1024
NB = 98
NPAD = NB * BLK
NTILE = 16
PT = NPAD // NTILE
CH = 128
NCHUNK = PT // CH
CBW = 16
NCB_PER_CORE = (HID // CBW) // 2
ZR = 256


def _stats1_body(feat_ref, w1t_ref, acc_ref):
    i = pl.program_id(0)
    x = jnp.dot(feat_ref[...].astype(jnp.bfloat16), w1t_ref[...],
                preferred_element_type=jnp.float32)
    s = jnp.sum(x, axis=0, keepdims=True)
    sq = jnp.sum(x * x, axis=0, keepdims=True)

    @pl.when(i == 0)
    def _():
        acc_ref[...] = jnp.zeros_like(acc_ref)

    acc_ref[0:1, :] += s
    acc_ref[1:2, :] += sq


def _fwd1_body(feat_ref, c0_ref, c1_ref, c2_ref, w1t_ref, a1_ref, b1_ref,
               y_ref, idx_ref, vm_ref):
    i = pl.program_id(0)
    x = jnp.dot(feat_ref[...].astype(jnp.bfloat16), w1t_ref[...],
                preferred_element_type=jnp.float32)
    y = jnp.maximum(x * a1_ref[...] + b1_ref[...], 0.0)
    rows = i * BLK + lax.broadcasted_iota(jnp.int32, (BLK, 1), 0)
    y_ref[...] = jnp.where(rows < N, y, 0.0)

    lanes = i * BLK + lax.broadcasted_iota(jnp.int32, (1, 1, BLK), 2)
    v0 = jnp.floor(c0_ref[...] / GRID).astype(jnp.int32)
    v1 = jnp.floor(c1_ref[...] / GRID).astype(jnp.int32)
    v2 = jnp.floor(c2_ref[...] / GRID).astype(jnp.int32)
    key = v0 * 10000 + v1 * 100 + v2
    idx = jnp.clip(jnp.mod(key, N), 0, N - 1)
    valid = lanes < N
    idx_ref[...] = jnp.where(valid, idx, 0)
    vm_ref[...] = jnp.where(valid, 1.0, 0.0)


def _fwd2_body(ps_ref, cnt_ref, w2t_ref, h_ref, acc_ref):
    i = pl.program_id(0)
    cnt = jnp.maximum(cnt_ref[...], 1.0)
    p = ps_ref[...] / cnt
    h = jnp.dot(p.astype(jnp.bfloat16), w2t_ref[...],
                preferred_element_type=jnp.float32)
    h_ref[...] = h.astype(jnp.bfloat16)
    s = jnp.sum(h, axis=0, keepdims=True)
    sq = jnp.sum(h * h, axis=0, keepdims=True)

    @pl.when(i == 0)
    def _():
        acc_ref[...] = jnp.zeros_like(acc_ref)

    acc_ref[0:1, :] += s
    acc_ref[1:2, :] += sq


def _out_body(h_ref, a2_ref, b2_ref, wot_ref, bout_ref, o_ref):
    h = h_ref[...].astype(jnp.float32)
    y2 = jnp.maximum(h * a2_ref[...] + b2_ref[...], 0.0)
    o_ref[...] = (jnp.dot(y2.astype(jnp.bfloat16), wot_ref[...],
                          preferred_element_type=jnp.float32)
                  + bout_ref[...])


def _full(shape):
    return pl.BlockSpec(shape, lambda i: (0,) * len(shape))


def _stats1(feat_pad, w1t):
    return pl.pallas_call(
        _stats1_body,
        grid=(NB,),
        in_specs=[pl.BlockSpec((BLK, IN_C), lambda i: (i, 0)), _full((IN_C, HID))],
        out_specs=_full((8, HID)),
        out_shape=jax.ShapeDtypeStruct((8, HID), jnp.float32),
    )(feat_pad, w1t)


def _fwd1(feat_pad, c0, c1, c2, w1t, a1, b1):
    return pl.pallas_call(
        _fwd1_body,
        grid=(NB,),
        in_specs=[
            pl.BlockSpec((BLK, IN_C), lambda i: (i, 0)),
            pl.BlockSpec((1, 1, BLK), lambda i: (i, 0, 0)),
            pl.BlockSpec((1, 1, BLK), lambda i: (i, 0, 0)),
            pl.BlockSpec((1, 1, BLK), lambda i: (i, 0, 0)),
            _full((IN_C, HID)),
            _full((1, HID)),
            _full((1, HID)),
        ],
        out_specs=[
            pl.BlockSpec((BLK, HID), lambda i: (i, 0)),
            pl.BlockSpec((1, 1, BLK), lambda i: (i, 0, 0)),
            pl.BlockSpec((1, 1, BLK), lambda i: (i, 0, 0)),
        ],
        out_shape=[
            jax.ShapeDtypeStruct((NPAD, HID), jnp.float32),
            jax.ShapeDtypeStruct((NB, 1, BLK), jnp.int32),
            jax.ShapeDtypeStruct((NB, 1, BLK), jnp.float32),
        ],
    )(feat_pad, c0, c1, c2, w1t, a1, b1)


def _fwd2(pooled, counts2d, w2t):
    return pl.pallas_call(
        _fwd2_body,
        grid=(NB,),
        in_specs=[
            pl.BlockSpec((BLK, HID), lambda i: (i, 0)),
            pl.BlockSpec((BLK, 1), lambda i: (i, 0)),
            _full((HID, HID)),
        ],
        out_specs=[
            pl.BlockSpec((BLK, HID), lambda i: (i, 0)),
            _full((8, HID)),
        ],
        out_shape=[
            jax.ShapeDtypeStruct((NPAD, HID), jnp.bfloat16),
            jax.ShapeDtypeStruct((8, HID), jnp.float32),
        ],
    )(pooled, counts2d, w2t)


def _out(h, a2, b2, wot, bout2):
    return pl.pallas_call(
        _out_body,
        grid=(NB,),
        in_specs=[
            pl.BlockSpec((BLK, HID), lambda i: (i, 0)),
            _full((1, HID)),
            _full((1, HID)),
            _full((HID, OUT_C)),
            _full((1, OUT_C)),
        ],
        out_specs=pl.BlockSpec((BLK, OUT_C), lambda i: (i, 0)),
        out_shape=jax.ShapeDtypeStruct((NPAD, OUT_C), jnp.float32),
    )(h, a2, b2, wot, bout2)


def _sc_body(y_hbm, idx_hbm, vm_hbm, z_hbm, z1_hbm,
             pooled_hbm, counts_hbm,
             acc, cacc, ibuf, ybuf0, ybuf1, zbuf, vbuf, rs0, rs1, ws):
    c = lax.axis_index("c")
    s = lax.axis_index("s")
    base = s * PT

    pltpu.sync_copy(idx_hbm.at[s], ibuf)
    pltpu.sync_copy(z_hbm, zbuf)

    @pl.when(c == 0)
    def _():
        pltpu.sync_copy(vm_hbm.at[s], vbuf)

    for j in range(NCB_PER_CORE):
        col = (c * NCB_PER_CORE + j) * CBW

        def _rd(k, buf, sem, col=col):
            return pltpu.async_copy(
                y_hbm.at[pl.ds(base + k * CH, CH), pl.ds(col, CBW)],
                buf, sem)

        _rd(0, ybuf0, rs0)
        _rd(1, ybuf1, rs1)
        for z in range(PT // ZR):
            pltpu.sync_copy(zbuf, acc.at[pl.ds(base + z * ZR, ZR), :])
        pltpu.sync_copy(zbuf.at[pl.ds(0, CH), :],
                        acc.at[pl.ds(base + (PT // ZR) * ZR, CH), :])
        plsc.subcore_barrier()

        def _pair(u, carry):
            k0 = 2 * u
            pltpu.make_async_copy(
                y_hbm.at[pl.ds(base + k0 * CH, CH), pl.ds(col, CBW)],
                ybuf0, rs0).wait()
            pltpu.sync_copy(ybuf0, acc.at[ibuf.at[k0]], add=True)
            _rd(k0 + 2, ybuf0, rs0)
            pltpu.make_async_copy(
                y_hbm.at[pl.ds(base + (k0 + 1) * CH, CH), pl.ds(col, CBW)],
                ybuf1, rs1).wait()
            pltpu.sync_copy(ybuf1, acc.at[ibuf.at[k0 + 1]], add=True)

            @pl.when(k0 + 3 < NCHUNK)
            def _():
                _rd(k0 + 3, ybuf1, rs1)

            return carry

        lax.fori_loop(0, (NCHUNK - 1) // 2, _pair, 0)
        pltpu.make_async_copy(
            y_hbm.at[pl.ds(base + (NCHUNK - 1) * CH, CH), pl.ds(col, CBW)],
            ybuf0, rs0).wait()
        pltpu.sync_copy(ybuf0, acc.at[ibuf.at[NCHUNK - 1]], add=True)
        plsc.subcore_barrier()

        for z in range(6):
            pltpu.async_copy(acc.at[pl.ds(base + z * BLK, BLK), :],
                             pooled_hbm.at[pl.ds(base + z * BLK, BLK),
                                           pl.ds(col, CBW)], ws)
        pltpu.async_copy(acc.at[pl.ds(base + 6 * BLK, CH), :],
                         pooled_hbm.at[pl.ds(base + 6 * BLK, CH),
                                       pl.ds(col, CBW)], ws)
        for z in range(6):
            pltpu.make_async_copy(acc.at[pl.ds(base + z * BLK, BLK), :],
                                  pooled_hbm.at[pl.ds(base + z * BLK, BLK),
                                                pl.ds(col, CBW)], ws).wait()
        pltpu.make_async_copy(acc.at[pl.ds(base + 6 * BLK, CH), :],
                              pooled_hbm.at[pl.ds(base + 6 * BLK, CH),
                                            pl.ds(col, CBW)], ws).wait()

    @pl.when(c == 0)
    def _():
        for z in range(6):
            pltpu.sync_copy(z1_hbm, cacc.at[pl.ds(base + z * BLK, BLK)])
        pltpu.sync_copy(z1_hbm.at[pl.ds(0, CH)],
                        cacc.at[pl.ds(base + 6 * BLK, CH)])
        plsc.subcore_barrier()

        def _cchunk(k, carry):
            pltpu.sync_copy(vbuf.at[k], cacc.at[ibuf.at[k]], add=True)
            return carry

        lax.fori_loop(0, NCHUNK, _cchunk, 0)
        plsc.subcore_barrier()
        pltpu.sync_copy(cacc.at[pl.ds(base, PT)], counts_hbm.at[pl.ds(base, PT)])


def _sc_scatter(y, idx3d, vm3d, zeros2d, zeros1d):
    kern = functools.partial(
        pl.kernel,
        out_type=[
            jax.ShapeDtypeStruct((NPAD, HID), jnp.float32),
            jax.ShapeDtypeStruct((NPAD,), jnp.float32),
        ],
        mesh=plsc.VectorSubcoreMesh(core_axis_name="c", subcore_axis_name="s"),
        compiler_params=pltpu.CompilerParams(use_tc_tiling_on_sc=False),
        scratch_types=[
            pltpu.VMEM_SHARED((NPAD, CBW), jnp.float32),
            pltpu.VMEM_SHARED((NPAD,), jnp.float32),
            pltpu.VMEM((NCHUNK, CH), jnp.int32),
            pltpu.VMEM((CH, CBW), jnp.float32),
            pltpu.VMEM((CH, CBW), jnp.float32),
            pltpu.VMEM((ZR, CBW), jnp.float32),
            pltpu.VMEM((NCHUNK, CH), jnp.float32),
            pltpu.SemaphoreType.DMA,
            pltpu.SemaphoreType.DMA,
            pltpu.SemaphoreType.DMA,
        ],
    )(_sc_body)
    return kern(y, idx3d, vm3d, zeros2d, zeros1d)


def kernel(feat, coord, offset, W1, gamma1, beta1, W2, gamma2, beta2, Wout, bout):
    del offset
    f32 = jnp.float32
    pad = NPAD - N
    feat_pad = jnp.pad(feat, ((0, pad), (0, 0)))
    c0 = jnp.pad(coord[:, 0], (0, pad)).reshape(NB, 1, BLK)
    c1 = jnp.pad(coord[:, 1], (0, pad)).reshape(NB, 1, BLK)
    c2 = jnp.pad(coord[:, 2], (0, pad)).reshape(NB, 1, BLK)
    w1t = W1.T.astype(jnp.bfloat16)
    w2t = W2.T.astype(jnp.bfloat16)
    wot = Wout.T.astype(jnp.bfloat16)
    eps = 1e-5

    st1 = _stats1(feat_pad, w1t)
    mean1 = st1[0] / N
    var1 = st1[1] / N - mean1 * mean1
    a1 = gamma1 / jnp.sqrt(var1 + eps)
    b1 = beta1 - mean1 * a1

    y, idx3, vm3 = _fwd1(feat_pad, c0, c1, c2, w1t,
                         a1.reshape(1, HID).astype(f32),
                         b1.reshape(1, HID).astype(f32))

    idx3d = idx3.reshape(NTILE, NCHUNK, CH)
    vm3d = vm3.reshape(NTILE, NCHUNK, CH)
    zeros2d = jnp.zeros((ZR, CBW), f32)
    zeros1d = jnp.zeros((BLK,), f32)

    pooled, counts = _sc_scatter(y, idx3d, vm3d, zeros2d, zeros1d)

    h, st2 = _fwd2(pooled, counts.reshape(NPAD, 1), w2t)
    mean2 = st2[0] / N
    var2 = st2[1] / N - mean2 * mean2
    a2 = gamma2 / jnp.sqrt(var2 + eps)
    b2 = beta2 - mean2 * a2

    logits = _out(h, a2.reshape(1, HID).astype(f32),
                  b2.reshape(1, HID).astype(f32),
                  wot, bout.reshape(1, OUT_C))
    return logits[:N]


# speedup vs baseline: 1.4052x; 1.1690x over previous
import functools

import jax
import jax.numpy as jnp
from jax import lax
from jax.experimental import pallas as pl
from jax.experimental.pallas import tpu as pltpu
from jax.experimental.pallas import tpu_sc as plsc

N = 100000
IN_C = 64
HID = 512
OUT_C = 13
GRID = 0.1

BLK = 1024
NB = 98
NPAD = NB * BLK
NTILE = 16
PT = NPAD // NTILE
CH = 128
NCHUNK = PT // CH
SLAB = 512
NSLAB = 12
CBW = 16
NCB_PER_CORE = (HID // CBW) // 2
ZR = 256
IROWS = NPAD // CH


def _stats1_body(feat_ref, w1t_ref, acc_ref):
    i = pl.program_id(0)
    x = jnp.dot(feat_ref[...].astype(jnp.bfloat16), w1t_ref[...],
                preferred_element_type=jnp.float32)
    rows = i * BLK + lax.broadcasted_iota(jnp.int32, (BLK, 1), 0)
    x = jnp.where(rows < N, x, 0.0)
    s = jnp.sum(x, axis=0, keepdims=True)
    sq = jnp.sum(x * x, axis=0, keepdims=True)

    @pl.when(i == 0)
    def _():
        acc_ref[...] = jnp.zeros_like(acc_ref)

    acc_ref[0:1, :] += s
    acc_ref[1:2, :] += sq


def _fwd1_body(feat_ref, c0_ref, c1_ref, c2_ref, w1t_ref, a1_ref, b1_ref,
               y_ref, idx_ref):
    i = pl.program_id(0)
    x = jnp.dot(feat_ref[...].astype(jnp.bfloat16), w1t_ref[...],
                preferred_element_type=jnp.float32)
    y = jnp.maximum(x * a1_ref[...] + b1_ref[...], 0.0)
    rows = i * BLK + lax.broadcasted_iota(jnp.int32, (BLK, 1), 0)
    y_ref[...] = jnp.where(rows < N, y, 0.0)

    p = (i * BLK
         + 128 * lax.broadcasted_iota(jnp.int32, (8, 128), 0)
         + lax.broadcasted_iota(jnp.int32, (8, 128), 1))
    v0 = jnp.floor(c0_ref[...] / GRID).astype(jnp.int32)
    v1 = jnp.floor(c1_ref[...] / GRID).astype(jnp.int32)
    v2 = jnp.floor(c2_ref[...] / GRID).astype(jnp.int32)
    key = v0 * 10000 + v1 * 100 + v2
    idx = jnp.clip(jnp.mod(key, N), 0, N - 1)
    idx_ref[...] = jnp.where(p < N, idx, 0)


def _fwd2_body(ps_ref, cnt_ref, w2t_ref, h_ref, acc_ref):
    i = pl.program_id(0)
    cnt = cnt_ref[...]
    rows = i * BLK + lax.broadcasted_iota(jnp.int32, (BLK, 1), 0)
    cnt = cnt - jnp.where(rows == 0, float(NPAD - N), 0.0)
    cnt = jnp.maximum(cnt, 1.0)
    ph = ps_ref[...] / cnt
    h = jnp.dot(ph.astype(jnp.bfloat16), w2t_ref[...],
                preferred_element_type=jnp.float32)
    h = jnp.where(rows < N, h, 0.0)
    h_ref[...] = h.astype(jnp.bfloat16)
    s = jnp.sum(h, axis=0, keepdims=True)
    sq = jnp.sum(h * h, axis=0, keepdims=True)

    @pl.when(i == 0)
    def _():
        acc_ref[...] = jnp.zeros_like(acc_ref)

    acc_ref[0:1, :] += s
    acc_ref[1:2, :] += sq


def _out_body(h_ref, a2_ref, b2_ref, wot_ref, bout_ref, o_ref):
    h = h_ref[...].astype(jnp.float32)
    y2 = jnp.maximum(h * a2_ref[...] + b2_ref[...], 0.0)
    o_ref[...] = (jnp.dot(y2.astype(jnp.bfloat16), wot_ref[...],
                          preferred_element_type=jnp.float32)
                  + bout_ref[...])


def _full(shape):
    return pl.BlockSpec(shape, lambda i: (0,) * len(shape))


def _stats1(feat, w1t):
    return pl.pallas_call(
        _stats1_body,
        grid=(NB,),
        in_specs=[pl.BlockSpec((BLK, IN_C), lambda i: (i, 0)), _full((IN_C, HID))],
        out_specs=_full((8, HID)),
        out_shape=jax.ShapeDtypeStruct((8, HID), jnp.float32),
    )(feat, w1t)


def _fwd1(feat, c0, c1, c2, w1t, a1, b1):
    return pl.pallas_call(
        _fwd1_body,
        grid=(NB,),
        in_specs=[
            pl.BlockSpec((BLK, IN_C), lambda i: (i, 0)),
            pl.BlockSpec((8, CH), lambda i: (i, 0)),
            pl.BlockSpec((8, CH), lambda i: (i, 0)),
            pl.BlockSpec((8, CH), lambda i: (i, 0)),
            _full((IN_C, HID)),
            _full((1, HID)),
            _full((1, HID)),
        ],
        out_specs=[
            pl.BlockSpec((BLK, HID), lambda i: (i, 0)),
            pl.BlockSpec((8, CH), lambda i: (i, 0)),
        ],
        out_shape=[
            jax.ShapeDtypeStruct((NPAD, HID), jnp.float32),
            jax.ShapeDtypeStruct((IROWS, CH), jnp.int32),
        ],
    )(feat, c0, c1, c2, w1t, a1, b1)


def _fwd2(pooled, counts2d, w2t):
    return pl.pallas_call(
        _fwd2_body,
        grid=(NB,),
        in_specs=[
            pl.BlockSpec((BLK, HID), lambda i: (i, 0)),
            pl.BlockSpec((BLK, 1), lambda i: (i, 0)),
            _full((HID, HID)),
        ],
        out_specs=[
            pl.BlockSpec((BLK, HID), lambda i: (i, 0)),
            _full((8, HID)),
        ],
        out_shape=[
            jax.ShapeDtypeStruct((NPAD, HID), jnp.bfloat16),
            jax.ShapeDtypeStruct((8, HID), jnp.float32),
        ],
    )(pooled, counts2d, w2t)


def _out(h, a2, b2, wot, bout2):
    return pl.pallas_call(
        _out_body,
        grid=(NB,),
        in_specs=[
            pl.BlockSpec((BLK, HID), lambda i: (i, 0)),
            _full((1, HID)),
            _full((1, HID)),
            _full((HID, OUT_C)),
            _full((1, OUT_C)),
        ],
        out_specs=pl.BlockSpec((BLK, OUT_C), lambda i: (i, 0)),
        out_shape=jax.ShapeDtypeStruct((N, OUT_C), jnp.float32),
    )(h, a2, b2, wot, bout2)


def _sc_body(y_hbm, idx_hbm, z_hbm, z1_hbm,
             pooled_hbm, counts_hbm,
             acc, cacc, iring, yb0, yb1, zbuf, cbuf,
             rs0, rs1, is0, is1, ws):
    c = lax.axis_index("c")
    s = lax.axis_index("s")
    base = s * PT
    irow = s * NCHUNK

    pltpu.sync_copy(z_hbm, zbuf)

    def _rd_y(t, buf, sem, col):
        return pltpu.async_copy(
            y_hbm.at[pl.ds(base + t * SLAB, SLAB), pl.ds(col, CBW)], buf, sem)

    def _wait_y(t, buf, sem, col):
        pltpu.make_async_copy(
            y_hbm.at[pl.ds(base + t * SLAB, SLAB), pl.ds(col, CBW)],
            buf, sem).wait()

    def _rd_i(t, b, sem):
        return pltpu.async_copy(
            idx_hbm.at[pl.ds(irow + t * 4, 4), :], iring.at[b], sem)

    def _wait_i(t, b, sem):
        pltpu.make_async_copy(
            idx_hbm.at[pl.ds(irow + t * 4, 4), :], iring.at[b], sem).wait()

    for j in range(NCB_PER_CORE):
        col = (c * NCB_PER_CORE + j) * CBW

        _rd_i(0, 0, is0)
        _rd_y(0, yb0, rs0, col)
        _rd_i(1, 1, is1)
        _rd_y(1, yb1, rs1, col)
        for z in range(PT // ZR):
            pltpu.async_copy(zbuf, acc.at[pl.ds(base + z * ZR, ZR), :], ws)
        pltpu.async_copy(zbuf.at[pl.ds(0, CH), :],
                         acc.at[pl.ds(base + (PT // ZR) * ZR, CH), :], ws)
        for z in range(PT // ZR):
            pltpu.make_async_copy(
                zbuf, acc.at[pl.ds(base + z * ZR, ZR), :], ws).wait()
        pltpu.make_async_copy(
            zbuf.at[pl.ds(0, CH), :],
            acc.at[pl.ds(base + (PT // ZR) * ZR, CH), :], ws).wait()
        plsc.subcore_barrier()

        def _pair(u, carry, col=col):
            t0 = 2 * u
            _wait_i(t0, 0, is0)
            _wait_y(t0, yb0, rs0, col)
            for d in range(4):
                pltpu.sync_copy(yb0.at[pl.ds(d * CH, CH), :],
                                acc.at[iring.at[0, d]], add=True)

            @pl.when(t0 + 2 < NSLAB)
            def _():
                _rd_i(t0 + 2, 0, is0)
                _rd_y(t0 + 2, yb0, rs0, col)

            _wait_i(t0 + 1, 1, is1)
            _wait_y(t0 + 1, yb1, rs1, col)
            for d in range(4):
                pltpu.sync_copy(yb1.at[pl.ds(d * CH, CH), :],
                                acc.at[iring.at[1, d]], add=True)

            @pl.when(t0 + 3 < NSLAB)
            def _():
                _rd_i(t0 + 3, 1, is1)
                _rd_y(t0 + 3, yb1, rs1, col)

            return carry

        lax.fori_loop(0, NSLAB // 2, _pair, 0)

        pltpu.sync_copy(idx_hbm.at[irow + NCHUNK - 1], iring.at[0, 0])
        pltpu.sync_copy(
            y_hbm.at[pl.ds(base + NSLAB * SLAB, CH), pl.ds(col, CBW)],
            yb0.at[pl.ds(0, CH), :])
        pltpu.sync_copy(yb0.at[pl.ds(0, CH), :],
                        acc.at[iring.at[0, 0]], add=True)
        plsc.subcore_barrier()

        for z in range(6):
            pltpu.async_copy(acc.at[pl.ds(base + z * BLK, BLK), :],
                             pooled_hbm.at[pl.ds(base + z * BLK, BLK),
                                           pl.ds(col, CBW)], ws)
        pltpu.async_copy(acc.at[pl.ds(base + 6 * BLK, CH), :],
                         pooled_hbm.at[pl.ds(base + 6 * BLK, CH),
                                       pl.ds(col, CBW)], ws)
        for z in range(6):
            pltpu.make_async_copy(acc.at[pl.ds(base + z * BLK, BLK), :],
                                  pooled_hbm.at[pl.ds(base + z * BLK, BLK),
                                                pl.ds(col, CBW)], ws).wait()
        pltpu.make_async_copy(acc.at[pl.ds(base + 6 * BLK, CH), :],
                              pooled_hbm.at[pl.ds(base + 6 * BLK, CH),
                                            pl.ds(col, CBW)], ws).wait()

    @pl.when(c == 0)
    def _():
        for q in range(8):
            cbuf[pl.ds(q * 16, 16)] = jnp.ones((16,), jnp.float32)
        for z in range(6):
            pltpu.async_copy(z1_hbm, cacc.at[pl.ds(base + z * BLK, BLK)], ws)
        pltpu.async_copy(z1_hbm.at[pl.ds(0, CH)],
                         cacc.at[pl.ds(base + 6 * BLK, CH)], ws)
        for z in range(6):
            pltpu.make_async_copy(
                z1_hbm, cacc.at[pl.ds(base + z * BLK, BLK)], ws).wait()
        pltpu.make_async_copy(z1_hbm.at[pl.ds(0, CH)],
                              cacc.at[pl.ds(base + 6 * BLK, CH)], ws).wait()
        plsc.subcore_barrier()

        _rd_i(0, 0, is0)
        _rd_i(1, 1, is1)

        def _cpair(u, carry):
            t0 = 2 * u
            _wait_i(t0, 0, is0)
            for d in range(4):
                pltpu.sync_copy(cbuf, cacc.at[iring.at[0, d]], add=True)

            @pl.when(t0 + 2 < NSLAB)
            def _():
                _rd_i(t0 + 2, 0, is0)

            _wait_i(t0 + 1, 1, is1)
            for d in range(4):
                pltpu.sync_copy(cbuf, cacc.at[iring.at[1, d]], add=True)

            @pl.when(t0 + 3 < NSLAB)
            def _():
                _rd_i(t0 + 3, 1, is1)

            return carry

        lax.fori_loop(0, NSLAB // 2, _cpair, 0)
        pltpu.sync_copy(idx_hbm.at[irow + NCHUNK - 1], iring.at[0, 0])
        pltpu.sync_copy(cbuf, cacc.at[iring.at[0, 0]], add=True)
        plsc.subcore_barrier()
        pltpu.sync_copy(cacc.at[pl.ds(base, PT)], counts_hbm.at[pl.ds(base, PT)])


def _sc_scatter(y, idx2d, zeros2d, zeros1d):
    kern = functools.partial(
        pl.kernel,
        out_type=[
            jax.ShapeDtypeStruct((NPAD, HID), jnp.float32),
            jax.ShapeDtypeStruct((NPAD,), jnp.float32),
        ],
        mesh=plsc.VectorSubcoreMesh(core_axis_name="c", subcore_axis_name="s"),
        compiler_params=pltpu.CompilerParams(use_tc_tiling_on_sc=False),
        scratch_types=[
            pltpu.VMEM_SHARED((NPAD, CBW), jnp.float32),
            pltpu.VMEM_SHARED((NPAD,), jnp.float32),
            pltpu.VMEM((2, 4, CH), jnp.int32),
            pltpu.VMEM((SLAB, CBW), jnp.float32),
            pltpu.VMEM((SLAB, CBW), jnp.float32),
            pltpu.VMEM((ZR, CBW), jnp.float32),
            pltpu.VMEM((CH,), jnp.float32),
            pltpu.SemaphoreType.DMA,
            pltpu.SemaphoreType.DMA,
            pltpu.SemaphoreType.DMA,
            pltpu.SemaphoreType.DMA,
            pltpu.SemaphoreType.DMA,
        ],
    )(_sc_body)
    return kern(y, idx2d, zeros2d, zeros1d)


def kernel(feat, coord, offset, W1, gamma1, beta1, W2, gamma2, beta2, Wout, bout):
    del offset
    f32 = jnp.float32
    pad = NPAD - N
    c0 = jnp.pad(coord[:, 0], (0, pad)).reshape(IROWS, CH)
    c1 = jnp.pad(coord[:, 1], (0, pad)).reshape(IROWS, CH)
    c2 = jnp.pad(coord[:, 2], (0, pad)).reshape(IROWS, CH)
    w1t = W1.T.astype(jnp.bfloat16)
    w2t = W2.T.astype(jnp.bfloat16)
    wot = Wout.T.astype(jnp.bfloat16)
    eps = 1e-5

    st1 = _stats1(feat, w1t)
    mean1 = st1[0] / N
    var1 = st1[1] / N - mean1 * mean1
    a1 = gamma1 / jnp.sqrt(var1 + eps)
    b1 = beta1 - mean1 * a1

    y, idx2d = _fwd1(feat, c0, c1, c2, w1t,
                     a1.reshape(1, HID).astype(f32),
                     b1.reshape(1, HID).astype(f32))

    zeros2d = jnp.zeros((ZR, CBW), f32)
    zeros1d = jnp.zeros((BLK,), f32)

    pooled, counts = _sc_scatter(y, idx2d, zeros2d, zeros1d)

    h, st2 = _fwd2(pooled, counts.reshape(NPAD, 1), w2t)
    mean2 = st2[0] / N
    var2 = st2[1] / N - mean2 * mean2
    a2 = gamma2 / jnp.sqrt(var2 + eps)
    b2 = beta2 - mean2 * a2

    return _out(h, a2.reshape(1, HID).astype(f32),
                b2.reshape(1, HID).astype(f32),
                wot, bout.reshape(1, OUT_C))


# speedup vs baseline: 1.4380x; 1.0234x over previous
import functools

import jax
import jax.numpy as jnp
from jax import lax
from jax.experimental import pallas as pl
from jax.experimental.pallas import tpu as pltpu
from jax.experimental.pallas import tpu_sc as plsc

N = 100000
IN_C = 64
HID = 512
OUT_C = 13
GRID = 0.1

BLK = 1024
NB = 98
NPAD = NB * BLK
NTILE = 16
PT = NPAD // NTILE
CH = 128
NCHUNK = PT // CH
SLAB = 512
NSLAB = 12
CBW = 16
NCB_PER_CORE = (HID // CBW) // 2
ZR = 256
IROWS = NPAD // CH


def _stats1_body(feat_ref, w1t_ref, acc_ref):
    i = pl.program_id(0)
    x = jnp.dot(feat_ref[...].astype(jnp.bfloat16), w1t_ref[...],
                preferred_element_type=jnp.float32)
    rows = i * BLK + lax.broadcasted_iota(jnp.int32, (BLK, 1), 0)
    x = jnp.where(rows < N, x, 0.0)
    s = jnp.sum(x, axis=0, keepdims=True)
    sq = jnp.sum(x * x, axis=0, keepdims=True)

    @pl.when(i == 0)
    def _():
        acc_ref[...] = jnp.zeros_like(acc_ref)

    acc_ref[0:1, :] += s
    acc_ref[1:2, :] += sq


def _fwd1_body(feat_ref, c0_ref, c1_ref, c2_ref, w1t_ref, a1_ref, b1_ref,
               y_ref, idx_ref):
    i = pl.program_id(0)
    x = jnp.dot(feat_ref[...].astype(jnp.bfloat16), w1t_ref[...],
                preferred_element_type=jnp.float32)
    y = jnp.maximum(x * a1_ref[...] + b1_ref[...], 0.0)
    rows = i * BLK + lax.broadcasted_iota(jnp.int32, (BLK, 1), 0)
    y_ref[...] = jnp.where(rows < N, y, 0.0)

    p = (i * BLK
         + 128 * lax.broadcasted_iota(jnp.int32, (8, 128), 0)
         + lax.broadcasted_iota(jnp.int32, (8, 128), 1))
    v0 = jnp.floor(c0_ref[...] / GRID).astype(jnp.int32)
    v1 = jnp.floor(c1_ref[...] / GRID).astype(jnp.int32)
    v2 = jnp.floor(c2_ref[...] / GRID).astype(jnp.int32)
    key = v0 * 10000 + v1 * 100 + v2
    idx = jnp.clip(jnp.mod(key, N), 0, N - 1)
    idx_ref[...] = jnp.where(p < N, idx, 0)


def _fwd2_body(ps_ref, cnt_ref, w2t_ref, h_ref, acc_ref):
    i = pl.program_id(0)
    cnt = cnt_ref[...][:, 0:1]
    rows = i * BLK + lax.broadcasted_iota(jnp.int32, (BLK, 1), 0)
    cnt = cnt - jnp.where(rows == 0, float(NPAD - N), 0.0)
    cnt = jnp.maximum(cnt, 1.0)
    ph = ps_ref[...] / cnt
    h = jnp.dot(ph.astype(jnp.bfloat16), w2t_ref[...],
                preferred_element_type=jnp.float32)
    h = jnp.where(rows < N, h, 0.0)
    h_ref[...] = h.astype(jnp.bfloat16)
    s = jnp.sum(h, axis=0, keepdims=True)
    sq = jnp.sum(h * h, axis=0, keepdims=True)

    @pl.when(i == 0)
    def _():
        acc_ref[...] = jnp.zeros_like(acc_ref)

    acc_ref[0:1, :] += s
    acc_ref[1:2, :] += sq


def _out_body(h_ref, a2_ref, b2_ref, wot_ref, bout_ref, o_ref):
    h = h_ref[...].astype(jnp.float32)
    y2 = jnp.maximum(h * a2_ref[...] + b2_ref[...], 0.0)
    o_ref[...] = (jnp.dot(y2.astype(jnp.bfloat16), wot_ref[...],
                          preferred_element_type=jnp.float32)
                  + bout_ref[...])


def _full(shape):
    return pl.BlockSpec(shape, lambda i: (0,) * len(shape))


def _stats1(feat, w1t):
    return pl.pallas_call(
        _stats1_body,
        grid=(NB,),
        in_specs=[pl.BlockSpec((BLK, IN_C), lambda i: (i, 0)), _full((IN_C, HID))],
        out_specs=_full((8, HID)),
        out_shape=jax.ShapeDtypeStruct((8, HID), jnp.float32),
    )(feat, w1t)


def _fwd1(feat, c0, c1, c2, w1t, a1, b1):
    return pl.pallas_call(
        _fwd1_body,
        grid=(NB,),
        in_specs=[
            pl.BlockSpec((BLK, IN_C), lambda i: (i, 0)),
            pl.BlockSpec((8, CH), lambda i: (i, 0)),
            pl.BlockSpec((8, CH), lambda i: (i, 0)),
            pl.BlockSpec((8, CH), lambda i: (i, 0)),
            _full((IN_C, HID)),
            _full((1, HID)),
            _full((1, HID)),
        ],
        out_specs=[
            pl.BlockSpec((BLK, HID), lambda i: (i, 0)),
            pl.BlockSpec((8, CH), lambda i: (i, 0)),
        ],
        out_shape=[
            jax.ShapeDtypeStruct((NPAD, HID), jnp.float32),
            jax.ShapeDtypeStruct((IROWS, CH), jnp.int32),
        ],
    )(feat, c0, c1, c2, w1t, a1, b1)


def _fwd2(pooled, counts2d, w2t):
    return pl.pallas_call(
        _fwd2_body,
        grid=(NB,),
        in_specs=[
            pl.BlockSpec((BLK, HID), lambda i: (i, 0)),
            pl.BlockSpec((BLK, CBW), lambda i: (i, 0)),
            _full((HID, HID)),
        ],
        out_specs=[
            pl.BlockSpec((BLK, HID), lambda i: (i, 0)),
            _full((8, HID)),
        ],
        out_shape=[
            jax.ShapeDtypeStruct((NPAD, HID), jnp.bfloat16),
            jax.ShapeDtypeStruct((8, HID), jnp.float32),
        ],
    )(pooled, counts2d, w2t)


def _out(h, a2, b2, wot, bout2):
    return pl.pallas_call(
        _out_body,
        grid=(NB,),
        in_specs=[
            pl.BlockSpec((BLK, HID), lambda i: (i, 0)),
            _full((1, HID)),
            _full((1, HID)),
            _full((HID, OUT_C)),
            _full((1, OUT_C)),
        ],
        out_specs=pl.BlockSpec((BLK, OUT_C), lambda i: (i, 0)),
        out_shape=jax.ShapeDtypeStruct((N, OUT_C), jnp.float32),
    )(h, a2, b2, wot, bout2)


def _sc_body(y_hbm, idx_hbm, z_hbm,
             pooled_hbm, counts_hbm,
             acc, iring, yb0, yb1, zbuf,
             rs0, rs1, is0, is1, ws):
    c = lax.axis_index("c")
    s = lax.axis_index("s")
    base = s * PT

    pltpu.sync_copy(z_hbm, zbuf)

    def _rd_y(t, buf, sem, col):
        return pltpu.async_copy(
            y_hbm.at[pl.ds(base + t * SLAB, SLAB), pl.ds(col, CBW)], buf, sem)

    def _wait_y(t, buf, sem, col):
        pltpu.make_async_copy(
            y_hbm.at[pl.ds(base + t * SLAB, SLAB), pl.ds(col, CBW)],
            buf, sem).wait()

    def _rd_i(t, b, sem):
        return pltpu.async_copy(
            idx_hbm.at[pl.ds(base + t * SLAB, SLAB)], iring.at[b], sem)

    def _wait_i(t, b, sem):
        pltpu.make_async_copy(
            idx_hbm.at[pl.ds(base + t * SLAB, SLAB)], iring.at[b], sem).wait()

    def _zero_own():
        for z in range(PT // ZR):
            pltpu.async_copy(zbuf, acc.at[pl.ds(base + z * ZR, ZR), :], ws)
        pltpu.async_copy(zbuf.at[pl.ds(0, CH), :],
                         acc.at[pl.ds(base + (PT // ZR) * ZR, CH), :], ws)
        for z in range(PT // ZR):
            pltpu.make_async_copy(
                zbuf, acc.at[pl.ds(base + z * ZR, ZR), :], ws).wait()
        pltpu.make_async_copy(
            zbuf.at[pl.ds(0, CH), :],
            acc.at[pl.ds(base + (PT // ZR) * ZR, CH), :], ws).wait()

    def _write_own(dst):
        for z in range(2):
            pltpu.async_copy(acc.at[pl.ds(base + z * (PT // 2), PT // 2), :],
                             dst.at[pl.ds(base + z * (PT // 2), PT // 2)], ws)
        for z in range(2):
            pltpu.make_async_copy(
                acc.at[pl.ds(base + z * (PT // 2), PT // 2), :],
                dst.at[pl.ds(base + z * (PT // 2), PT // 2)], ws).wait()

    for j in range(NCB_PER_CORE):
        col = (c * NCB_PER_CORE + j) * CBW

        _rd_i(0, 0, is0)
        _rd_y(0, yb0, rs0, col)
        _rd_i(1, 1, is1)
        _rd_y(1, yb1, rs1, col)
        _zero_own()
        plsc.subcore_barrier()

        def _pair(u, carry, col=col):
            t0 = 2 * u
            _wait_i(t0, 0, is0)
            _wait_y(t0, yb0, rs0, col)
            pltpu.sync_copy(yb0, acc.at[iring.at[0]], add=True)

            @pl.when(t0 + 2 < NSLAB)
            def _():
                _rd_i(t0 + 2, 0, is0)
                _rd_y(t0 + 2, yb0, rs0, col)

            _wait_i(t0 + 1, 1, is1)
            _wait_y(t0 + 1, yb1, rs1, col)
            pltpu.sync_copy(yb1, acc.at[iring.at[1]], add=True)

            @pl.when(t0 + 3 < NSLAB)
            def _():
                _rd_i(t0 + 3, 1, is1)
                _rd_y(t0 + 3, yb1, rs1, col)

            return carry

        lax.fori_loop(0, NSLAB // 2, _pair, 0)

        pltpu.sync_copy(idx_hbm.at[pl.ds(base + NSLAB * SLAB, CH)],
                        iring.at[0, pl.ds(0, CH)])
        pltpu.sync_copy(
            y_hbm.at[pl.ds(base + NSLAB * SLAB, CH), pl.ds(col, CBW)],
            yb0.at[pl.ds(0, CH), :])
        pltpu.sync_copy(yb0.at[pl.ds(0, CH), :],
                        acc.at[iring.at[0, pl.ds(0, CH)]], add=True)
        plsc.subcore_barrier()
        _write_own(pooled_hbm.at[:, pl.ds(col, CBW)])

    @pl.when(c == 0)
    def _():
        def _ones(t, carry):
            yb0[t] = jnp.ones((CBW,), jnp.float32)
            return carry

        lax.fori_loop(0, SLAB, _ones, 0)
        _zero_own()
        plsc.subcore_barrier()
        _rd_i(0, 0, is0)
        _rd_i(1, 1, is1)

        def _cpair(u, carry):
            t0 = 2 * u
            _wait_i(t0, 0, is0)
            pltpu.sync_copy(yb0, acc.at[iring.at[0]], add=True)

            @pl.when(t0 + 2 < NSLAB)
            def _():
                _rd_i(t0 + 2, 0, is0)

            _wait_i(t0 + 1, 1, is1)
            pltpu.sync_copy(yb0, acc.at[iring.at[1]], add=True)

            @pl.when(t0 + 3 < NSLAB)
            def _():
                _rd_i(t0 + 3, 1, is1)

            return carry

        lax.fori_loop(0, NSLAB // 2, _cpair, 0)
        pltpu.sync_copy(idx_hbm.at[pl.ds(base + NSLAB * SLAB, CH)],
                        iring.at[0, pl.ds(0, CH)])
        pltpu.sync_copy(yb0.at[pl.ds(0, CH), :],
                        acc.at[iring.at[0, pl.ds(0, CH)]], add=True)
        plsc.subcore_barrier()
        _write_own(counts_hbm)


def _sc_scatter(y, idx2d, zeros2d):
    kern = functools.partial(
        pl.kernel,
        out_type=[
            jax.ShapeDtypeStruct((NPAD, HID), jnp.float32),
            jax.ShapeDtypeStruct((NPAD, CBW), jnp.float32),
        ],
        mesh=plsc.VectorSubcoreMesh(core_axis_name="c", subcore_axis_name="s"),
        compiler_params=pltpu.CompilerParams(use_tc_tiling_on_sc=False),
        scratch_types=[
            pltpu.VMEM_SHARED((NPAD, CBW), jnp.float32),
            pltpu.VMEM((2, SLAB), jnp.int32),
            pltpu.VMEM((SLAB, CBW), jnp.float32),
            pltpu.VMEM((SLAB, CBW), jnp.float32),
            pltpu.VMEM((ZR, CBW), jnp.float32),
            pltpu.SemaphoreType.DMA,
            pltpu.SemaphoreType.DMA,
            pltpu.SemaphoreType.DMA,
            pltpu.SemaphoreType.DMA,
            pltpu.SemaphoreType.DMA,
        ],
    )(_sc_body)
    return kern(y, idx2d.reshape(NPAD), zeros2d)


def kernel(feat, coord, offset, W1, gamma1, beta1, W2, gamma2, beta2, Wout, bout):
    del offset
    f32 = jnp.float32
    pad = NPAD - N
    c0 = jnp.pad(coord[:, 0], (0, pad)).reshape(IROWS, CH)
    c1 = jnp.pad(coord[:, 1], (0, pad)).reshape(IROWS, CH)
    c2 = jnp.pad(coord[:, 2], (0, pad)).reshape(IROWS, CH)
    w1t = W1.T.astype(jnp.bfloat16)
    w2t = W2.T.astype(jnp.bfloat16)
    wot = Wout.T.astype(jnp.bfloat16)
    eps = 1e-5

    st1 = _stats1(feat, w1t)
    mean1 = st1[0] / N
    var1 = st1[1] / N - mean1 * mean1
    a1 = gamma1 / jnp.sqrt(var1 + eps)
    b1 = beta1 - mean1 * a1

    y, idx2d = _fwd1(feat, c0, c1, c2, w1t,
                     a1.reshape(1, HID).astype(f32),
                     b1.reshape(1, HID).astype(f32))

    zeros2d = jnp.zeros((ZR, CBW), f32)

    pooled, counts = _sc_scatter(y, idx2d, zeros2d)

    h, st2 = _fwd2(pooled, counts, w2t)
    mean2 = st2[0] / N
    var2 = st2[1] / N - mean2 * mean2
    a2 = gamma2 / jnp.sqrt(var2 + eps)
    b2 = beta2 - mean2 * a2

    return _out(h, a2.reshape(1, HID).astype(f32),
                b2.reshape(1, HID).astype(f32),
                wot, bout.reshape(1, OUT_C))
